# baseline (speedup 1.0000x reference)
import functools

import jax
import jax.numpy as jnp
from jax import lax
from jax.experimental import pallas as pl
from jax.experimental.pallas import tpu as pltpu
from jax.experimental.pallas import tpu_sc as plsc

N = 10000
E = 320000
D = 128
CLS = 10
EPS = 1e-12

NC = 2
NS = 16
NW = NC * NS
C = 64
NCH = 159
PT = NCH * C
EPD = NW * PT
NBUF = 3
NPAD = 10240
RPT = NPAD // NS


def _sc_prop_body(src_hbm, dst_hbm, w_hbm, table_hbm, zeros_hbm, out_hbm,
                  src_v, dst_v, wf_v, rows_v, acc, sem_w, sem_g, sem_s):
    c = lax.axis_index("c")
    s = lax.axis_index("s")
    wid = c * NS + s

    pltpu.sync_copy(zeros_hbm, acc.at[pl.ds(s * RPT, RPT)])

    pltpu.sync_copy(src_hbm.at[wid], src_v)
    pltpu.sync_copy(dst_hbm.at[wid], dst_v)

    plsc.subcore_barrier()

    def wf_start(j, b):
        pltpu.async_copy(w_hbm.at[wid, j], wf_v.at[b], sem_w.at[b])

    def wf_wait(j, b):
        pltpu.make_async_copy(w_hbm.at[wid, j], wf_v.at[b],
                              sem_w.at[b]).wait()

    def gather_start(j, b):
        pltpu.async_copy(table_hbm.at[src_v.at[j]], rows_v.at[b],
                         sem_g.at[b])

    def gather_wait(j, b):
        pltpu.make_async_copy(table_hbm.at[src_v.at[j]], rows_v.at[b],
                              sem_g.at[b]).wait()

    def scatter_start(j, b):
        pltpu.async_copy(rows_v.at[b], acc.at[dst_v.at[j]], sem_s.at[b],
                         add=True)

    def scatter_wait(j, b):
        pltpu.make_async_copy(rows_v.at[b], acc.at[dst_v.at[j]],
                              sem_s.at[b]).wait()

    for b in range(NBUF):
        wf_start(b, b)
        gather_start(b, b)

    def round_body(r, carry):
        for b in range(NBUF):
            j = r * NBUF + b
            gather_wait(j, b)
            wf_wait(j, b)

            for g in range(C // 16):
                w16 = wf_v[b, pl.ds(g * 16, 16)]
                for l in range(16):
                    wv = w16.at[jnp.full((16,), l, jnp.int32)].get(
                        mode="promise_in_bounds")
                    for k in range(D // 16):
                        seg = rows_v[b, g * 16 + l, pl.ds(k * 16, 16)]
                        rows_v[b, g * 16 + l, pl.ds(k * 16, 16)] = seg * wv

            @pl.when(j + NBUF < NCH)
            def _(b=b, j=j):
                wf_start(j + NBUF, b)

            bp = (b - 1) % NBUF
            cond = (j - 1 + NBUF < NCH) if b != 0 else \
                jnp.logical_and(r > 0, j - 1 + NBUF < NCH)

            @pl.when(cond)
            def _(bp=bp, j=j):
                scatter_wait(j - 1, bp)
                gather_start(j - 1 + NBUF, bp)

            scatter_start(j, b)

        return carry

    lax.fori_loop(0, NCH // NBUF, round_body, 0)

    for b in range(NBUF):
        scatter_wait(NCH - NBUF + b, b)

    plsc.subcore_barrier()

    pltpu.sync_copy(acc.at[pl.ds(s * RPT, RPT)],
                    out_hbm.at[c].at[pl.ds(s * RPT, RPT)])


def _propagate(src3, dst3, w3, table, zeros):
    mesh = plsc.VectorSubcoreMesh(core_axis_name="c", subcore_axis_name="s",
                                  num_cores=NC, num_subcores=NS)
    return pl.kernel(
        _sc_prop_body,
        out_type=jax.ShapeDtypeStruct((NC, NPAD, D), jnp.float32),
        mesh=mesh,
        scratch_types=[
            pltpu.VMEM((NCH, C), jnp.int32),
            pltpu.VMEM((NCH, C), jnp.int32),
            pltpu.VMEM((NBUF, C), jnp.float32),
            pltpu.VMEM((NBUF, C, D), jnp.float32),
            pltpu.VMEM_SHARED((NPAD, D), jnp.float32),
            pltpu.SemaphoreType.DMA((NBUF,)),
            pltpu.SemaphoreType.DMA((NBUF,)),
            pltpu.SemaphoreType.DMA((NBUF,)),
        ],
        compiler_params=pltpu.CompilerParams(use_tc_tiling_on_sc=False),
    )(src3, dst3, w3, table, zeros)


_BR = 1000


def _mm3_body(h_ref, wa_ref, wb_ref, wc_ref, a_ref, b_ref, c_ref):
    h = h_ref[...]
    dn = (((1,), (1,)), ((), ()))
    a_ref[...] = lax.dot_general(h, wa_ref[...], dn,
                                 preferred_element_type=jnp.float32)
    b_ref[...] = lax.dot_general(h, wb_ref[...], dn,
                                 preferred_element_type=jnp.float32)
    c_ref[...] = lax.dot_general(h, wc_ref[...], dn,
                                 preferred_element_type=jnp.float32)


def _mm3(h, wa, wb, wc):
    full = pl.BlockSpec((D, D), lambda i: (0, 0))
    return pl.pallas_call(
        _mm3_body,
        grid=(N // _BR,),
        in_specs=[pl.BlockSpec((_BR, D), lambda i: (i, 0)), full, full, full],
        out_specs=[pl.BlockSpec((_BR, D), lambda i: (i, 0))] * 3,
        out_shape=[jax.ShapeDtypeStruct((N, D), jnp.float32)] * 3,
    )(h, wa, wb, wc)


def _combine(pin_ref, pout_ref, sh_ref, bmi_ref, bmo_ref, bsi_ref, bso_ref,
             cin_ref, cout_ref):
    pi = pin_ref[0] + pin_ref[1]
    po = pout_ref[0] + pout_ref[1]
    sh = sh_ref[...]
    ic = pi + bmi_ref[...] + sh + bsi_ref[...]
    oc = po + bmo_ref[...] + sh + bso_ref[...]
    return jnp.tanh(cin_ref[...] * ic + cout_ref[...] * oc)


def _layer_mid_body(pin_ref, pout_ref, sh_ref, bmi_ref, bmo_ref, bsi_ref,
                    bso_ref, cin_ref, cout_ref, wa_ref, wb_ref, wc_ref,
                    a_ref, b_ref, c_ref):
    h = _combine(pin_ref, pout_ref, sh_ref, bmi_ref, bmo_ref, bsi_ref,
                 bso_ref, cin_ref, cout_ref)
    dn = (((1,), (1,)), ((), ()))
    a_ref[...] = lax.dot_general(h, wa_ref[...], dn,
                                 preferred_element_type=jnp.float32)
    b_ref[...] = lax.dot_general(h, wb_ref[...], dn,
                                 preferred_element_type=jnp.float32)
    c_ref[...] = lax.dot_general(h, wc_ref[...], dn,
                                 preferred_element_type=jnp.float32)


def _combine_specs():
    return [
        pl.BlockSpec((NC, _BR, D), lambda i: (0, i, 0)),
        pl.BlockSpec((NC, _BR, D), lambda i: (0, i, 0)),
        pl.BlockSpec((_BR, D), lambda i: (i, 0)),
        pl.BlockSpec((1, D), lambda i: (0, 0)),
        pl.BlockSpec((1, D), lambda i: (0, 0)),
        pl.BlockSpec((1, D), lambda i: (0, 0)),
        pl.BlockSpec((1, D), lambda i: (0, 0)),
        pl.BlockSpec((_BR, 1), lambda i: (i, 0)),
        pl.BlockSpec((_BR, 1), lambda i: (i, 0)),
    ]


def _layer_mid(pin, pout, sh, bmi, bmo, bsi, bso, cin, cout, wa, wb, wc):
    full = pl.BlockSpec((D, D), lambda i: (0, 0))
    return pl.pallas_call(
        _layer_mid_body,
        grid=(N // _BR,),
        in_specs=_combine_specs() + [full, full, full],
        out_specs=[pl.BlockSpec((_BR, D), lambda i: (i, 0))] * 3,
        out_shape=[jax.ShapeDtypeStruct((N, D), jnp.float32)] * 3,
    )(pin, pout, sh, bmi, bmo, bsi, bso, cin, cout, wa, wb, wc)


def _layer_final_body(pin_ref, pout_ref, sh_ref, bmi_ref, bmo_ref, bsi_ref,
                      bso_ref, cin_ref, cout_ref, wdec_ref, bdec_ref,
                      logp_ref, norm_ref):
    h = _combine(pin_ref, pout_ref, sh_ref, bmi_ref, bmo_ref, bsi_ref,
                 bso_ref, cin_ref, cout_ref)
    s2 = jnp.sum(h * h, axis=1, keepdims=True)
    norm_ref[...] = h / (jnp.sqrt(s2) + EPS)
    dn = (((1,), (1,)), ((), ()))
    logits = lax.dot_general(h, wdec_ref[...], dn,
                             preferred_element_type=jnp.float32)
    logits = logits + bdec_ref[...]
    mask = lax.broadcasted_iota(jnp.int32, (_BR, D), 1) < CLS
    masked = jnp.where(mask, logits, -jnp.inf)
    m = jnp.max(masked, axis=1, keepdims=True)
    ex = jnp.where(mask, jnp.exp(logits - m), 0.0)
    ssum = jnp.sum(ex, axis=1, keepdims=True)
    logp_ref[...] = logits - m - jnp.log(ssum)


def _layer_final(pin, pout, sh, bmi, bmo, bsi, bso, cin, cout, wdec, bdec):
    full = pl.BlockSpec((D, D), lambda i: (0, 0))
    bspec = pl.BlockSpec((1, D), lambda i: (0, 0))
    return pl.pallas_call(
        _layer_final_body,
        grid=(N // _BR,),
        in_specs=_combine_specs() + [full, bspec],
        out_specs=[pl.BlockSpec((_BR, D), lambda i: (i, 0))] * 2,
        out_shape=[jax.ShapeDtypeStruct((N, D), jnp.float32)] * 2,
    )(pin, pout, sh, bmi, bmo, bsi, bso, cin, cout, wdec, bdec)


def kernel(x, edge_index_in, edge_weight_in, edge_index_out, edge_weight_out,
           Wmi0, Wmo0, Ws0, bmi0, bmo0, bsi0, bso0, Cin0, Cout0,
           Wmi1, Wmo1, Ws1, bmi1, bmo1, bsi1, bso1, Cin1, Cout1,
           W_dec, b_dec):
    ipad = jnp.zeros((EPD - E,), jnp.int32)
    fpad = jnp.zeros((EPD - E,), jnp.float32)
    src_in = jnp.concatenate([edge_index_in[0], ipad]).reshape(NW, NCH, C)
    dst_in = jnp.concatenate([edge_index_in[1], ipad]).reshape(NW, NCH, C)
    w_in = jnp.concatenate([edge_weight_in, fpad]).reshape(NW, NCH, C)
    src_out = jnp.concatenate([edge_index_out[0], ipad]).reshape(NW, NCH, C)
    dst_out = jnp.concatenate([edge_index_out[1], ipad]).reshape(NW, NCH, C)
    w_out = jnp.concatenate([edge_weight_out, fpad]).reshape(NW, NCH, C)
    zeros = jnp.zeros((RPT, D), jnp.float32)

    bmi0r, bmo0r = bmi0.reshape(1, D), bmo0.reshape(1, D)
    bsi0r, bso0r = bsi0.reshape(1, D), bso0.reshape(1, D)
    bmi1r, bmo1r = bmi1.reshape(1, D), bmo1.reshape(1, D)
    bsi1r, bso1r = bsi1.reshape(1, D), bso1.reshape(1, D)
    wdec_pad = jnp.zeros((D, D), jnp.float32).at[:CLS].set(W_dec)
    bdec_pad = jnp.zeros((1, D), jnp.float32).at[0, :CLS].set(b_dec)

    hmi0, hmo0, sh0 = _mm3(x, Wmi0, Wmo0, Ws0)
    pin0 = _propagate(src_in, dst_in, w_in, hmi0, zeros)
    pout0 = _propagate(src_out, dst_out, w_out, hmo0, zeros)

    hmi1, hmo1, sh1 = _layer_mid(pin0, pout0, sh0, bmi0r, bmo0r, bsi0r,
                                 bso0r, Cin0, Cout0, Wmi1, Wmo1, Ws1)
    pin1 = _propagate(src_in, dst_in, w_in, hmi1, zeros)
    pout1 = _propagate(src_out, dst_out, w_out, hmo1, zeros)

    logp_pad, norm_emb = _layer_final(pin1, pout1, sh1, bmi1r, bmo1r, bsi1r,
                                      bso1r, Cin1, Cout1, wdec_pad, bdec_pad)
    return (logp_pad[:, :CLS], norm_emb)


# speedup vs baseline: 1.8748x; 1.8748x over previous
import functools

import jax
import jax.numpy as jnp
from jax import lax
from jax.experimental import pallas as pl
from jax.experimental.pallas import tpu as pltpu
from jax.experimental.pallas import tpu_sc as plsc

N = 10000
E = 320000
D = 128
CLS = 10
EPS = 1e-12

NC = 2
NS = 16
DH = D // NC
C = 80
PT = E // NS
NCH = PT // C
NBUF = 5
NPAD = 10240
RPT = NPAD // NS


def _sc_prop_body(src_hbm, dst_hbm, w_hbm, table_hbm, zeros_hbm, out_hbm,
                  src_v, dst_v, w_v, rows_v, acc, sem_g, sem_s):
    c = lax.axis_index("c")
    s = lax.axis_index("s")

    def phase_body(p, carry0):
        pltpu.sync_copy(zeros_hbm, acc.at[pl.ds(s * RPT, RPT)])

        pltpu.sync_copy(src_hbm.at[p, s], src_v)
        pltpu.sync_copy(dst_hbm.at[p, s], dst_v)
        pltpu.sync_copy(w_hbm.at[p, s], w_v)

        plsc.subcore_barrier()

        def gather_start(j, b):
            pltpu.async_copy(table_hbm.at[p, c].at[src_v.at[j]],
                             rows_v.at[b], sem_g.at[b])

        def gather_wait(j, b):
            pltpu.make_async_copy(table_hbm.at[p, c].at[src_v.at[j]],
                                  rows_v.at[b], sem_g.at[b]).wait()

        def scatter_start(j, b):
            pltpu.async_copy(rows_v.at[b], acc.at[dst_v.at[j]], sem_s.at[b],
                             add=True)

        def scatter_wait(j, b):
            pltpu.make_async_copy(rows_v.at[b], acc.at[dst_v.at[j]],
                                  sem_s.at[b]).wait()

        for b in range(NBUF):
            gather_start(b, b)

        def round_body(r, carry):
            for b in range(NBUF):
                j = r * NBUF + b
                gather_wait(j, b)

                for g in range(C // 16):
                    w16 = w_v[pl.ds(j * C + g * 16, 16)]
                    for l in range(16):
                        wv = w16.at[jnp.full((16,), l, jnp.int32)].get(
                            mode="promise_in_bounds")
                        for k in range(DH // 16):
                            seg = rows_v[b, g * 16 + l, pl.ds(k * 16, 16)]
                            rows_v[b, g * 16 + l, pl.ds(k * 16, 16)] = \
                                seg * wv

                bp = (b - 1) % NBUF
                cond = (j - 1 + NBUF < NCH) if b != 0 else \
                    jnp.logical_and(r > 0, j - 1 + NBUF < NCH)

                @pl.when(cond)
                def _(bp=bp, j=j):
                    scatter_wait(j - 1, bp)
                    gather_start(j - 1 + NBUF, bp)

                scatter_start(j, b)

            return carry

        lax.fori_loop(0, NCH // NBUF, round_body, 0)

        for b in range(NBUF):
            scatter_wait(NCH - NBUF + b, b)

        plsc.subcore_barrier()

        pltpu.sync_copy(acc.at[pl.ds(s * RPT, RPT)],
                        out_hbm.at[p, c].at[pl.ds(s * RPT, RPT)])

        plsc.subcore_barrier()
        return carry0

    lax.fori_loop(0, 2, phase_body, 0)


def _propagate2(src4, dst4, w4, tables, zeros):
    mesh = plsc.VectorSubcoreMesh(core_axis_name="c", subcore_axis_name="s",
                                  num_cores=NC, num_subcores=NS)
    return pl.kernel(
        _sc_prop_body,
        out_type=jax.ShapeDtypeStruct((2, NC, NPAD, DH), jnp.float32),
        mesh=mesh,
        scratch_types=[
            pltpu.VMEM((NCH, C), jnp.int32),
            pltpu.VMEM((NCH, C), jnp.int32),
            pltpu.VMEM((PT,), jnp.float32),
            pltpu.VMEM((NBUF, C, DH), jnp.float32),
            pltpu.VMEM_SHARED((NPAD, DH), jnp.float32),
            pltpu.SemaphoreType.DMA((NBUF,)),
            pltpu.SemaphoreType.DMA((NBUF,)),
        ],
        compiler_params=pltpu.CompilerParams(use_tc_tiling_on_sc=False),
    )(src4, dst4, w4, tables, zeros)


_BR = 1000


def _store_halves(ref, val):
    ref[0] = val[:, :DH]
    ref[1] = val[:, DH:]


def _mm3_body(h_ref, wa_ref, wb_ref, wc_ref, tbl_ref, c_ref):
    h = h_ref[...]
    dn = (((1,), (1,)), ((), ()))
    _store_halves(tbl_ref.at[0], lax.dot_general(
        h, wa_ref[...], dn, preferred_element_type=jnp.float32))
    _store_halves(tbl_ref.at[1], lax.dot_general(
        h, wb_ref[...], dn, preferred_element_type=jnp.float32))
    c_ref[...] = lax.dot_general(h, wc_ref[...], dn,
                                 preferred_element_type=jnp.float32)


def _mm3(h, wa, wb, wc):
    full = pl.BlockSpec((D, D), lambda i: (0, 0))
    tspec = pl.BlockSpec((2, NC, _BR, DH), lambda i: (0, 0, i, 0))
    return pl.pallas_call(
        _mm3_body,
        grid=(N // _BR,),
        in_specs=[pl.BlockSpec((_BR, D), lambda i: (i, 0)), full, full, full],
        out_specs=[tspec, pl.BlockSpec((_BR, D), lambda i: (i, 0))],
        out_shape=[jax.ShapeDtypeStruct((2, NC, N, DH), jnp.float32),
                   jax.ShapeDtypeStruct((N, D), jnp.float32)],
    )(h, wa, wb, wc)


def _combine(pp_ref, sh_ref, bmi_ref, bmo_ref, bsi_ref, bso_ref,
             cin_ref, cout_ref):
    pi = jnp.concatenate([pp_ref[0, 0], pp_ref[0, 1]], axis=1)
    po = jnp.concatenate([pp_ref[1, 0], pp_ref[1, 1]], axis=1)
    sh = sh_ref[...]
    ic = pi + bmi_ref[...] + sh + bsi_ref[...]
    oc = po + bmo_ref[...] + sh + bso_ref[...]
    return jnp.tanh(cin_ref[...] * ic + cout_ref[...] * oc)


def _layer_mid_body(pin_ref, sh_ref, bmi_ref, bmo_ref, bsi_ref,
                    bso_ref, cin_ref, cout_ref, wa_ref, wb_ref, wc_ref,
                    tbl_ref, c_ref):
    h = _combine(pin_ref, sh_ref, bmi_ref, bmo_ref, bsi_ref,
                 bso_ref, cin_ref, cout_ref)
    dn = (((1,), (1,)), ((), ()))
    _store_halves(tbl_ref.at[0], lax.dot_general(
        h, wa_ref[...], dn, preferred_element_type=jnp.float32))
    _store_halves(tbl_ref.at[1], lax.dot_general(
        h, wb_ref[...], dn, preferred_element_type=jnp.float32))
    c_ref[...] = lax.dot_general(h, wc_ref[...], dn,
                                 preferred_element_type=jnp.float32)


def _combine_specs():
    return [
        pl.BlockSpec((2, NC, _BR, DH), lambda i: (0, 0, i, 0)),
        pl.BlockSpec((_BR, D), lambda i: (i, 0)),
        pl.BlockSpec((1, D), lambda i: (0, 0)),
        pl.BlockSpec((1, D), lambda i: (0, 0)),
        pl.BlockSpec((1, D), lambda i: (0, 0)),
        pl.BlockSpec((1, D), lambda i: (0, 0)),
        pl.BlockSpec((_BR, 1), lambda i: (i, 0)),
        pl.BlockSpec((_BR, 1), lambda i: (i, 0)),
    ]


def _layer_mid(pp, sh, bmi, bmo, bsi, bso, cin, cout, wa, wb, wc):
    full = pl.BlockSpec((D, D), lambda i: (0, 0))
    tspec = pl.BlockSpec((2, NC, _BR, DH), lambda i: (0, 0, i, 0))
    return pl.pallas_call(
        _layer_mid_body,
        grid=(N // _BR,),
        in_specs=_combine_specs() + [full, full, full],
        out_specs=[tspec, pl.BlockSpec((_BR, D), lambda i: (i, 0))],
        out_shape=[jax.ShapeDtypeStruct((2, NC, N, DH), jnp.float32),
                   jax.ShapeDtypeStruct((N, D), jnp.float32)],
    )(pp, sh, bmi, bmo, bsi, bso, cin, cout, wa, wb, wc)


def _layer_final_body(pin_ref, sh_ref, bmi_ref, bmo_ref, bsi_ref,
                      bso_ref, cin_ref, cout_ref, wdec_ref, bdec_ref,
                      logp_ref, norm_ref):
    h = _combine(pin_ref, sh_ref, bmi_ref, bmo_ref, bsi_ref,
                 bso_ref, cin_ref, cout_ref)
    s2 = jnp.sum(h * h, axis=1, keepdims=True)
    norm_ref[...] = h / (jnp.sqrt(s2) + EPS)
    dn = (((1,), (1,)), ((), ()))
    logits = lax.dot_general(h, wdec_ref[...], dn,
                             preferred_element_type=jnp.float32)
    logits = logits + bdec_ref[...]
    mask = lax.broadcasted_iota(jnp.int32, (_BR, D), 1) < CLS
    masked = jnp.where(mask, logits, -jnp.inf)
    m = jnp.max(masked, axis=1, keepdims=True)
    ex = jnp.where(mask, jnp.exp(logits - m), 0.0)
    ssum = jnp.sum(ex, axis=1, keepdims=True)
    logp_ref[...] = logits - m - jnp.log(ssum)


def _layer_final(pp, sh, bmi, bmo, bsi, bso, cin, cout, wdec, bdec):
    full = pl.BlockSpec((D, D), lambda i: (0, 0))
    bspec = pl.BlockSpec((1, D), lambda i: (0, 0))
    return pl.pallas_call(
        _layer_final_body,
        grid=(N // _BR,),
        in_specs=_combine_specs() + [full, bspec],
        out_specs=[pl.BlockSpec((_BR, D), lambda i: (i, 0))] * 2,
        out_shape=[jax.ShapeDtypeStruct((N, D), jnp.float32)] * 2,
    )(pp, sh, bmi, bmo, bsi, bso, cin, cout, wdec, bdec)


def kernel(x, edge_index_in, edge_weight_in, edge_index_out, edge_weight_out,
           Wmi0, Wmo0, Ws0, bmi0, bmo0, bsi0, bso0, Cin0, Cout0,
           Wmi1, Wmo1, Ws1, bmi1, bmo1, bsi1, bso1, Cin1, Cout1,
           W_dec, b_dec):
    src_b = jnp.stack([edge_index_in[0].reshape(NS, NCH, C),
                       edge_index_out[0].reshape(NS, NCH, C)])
    dst_b = jnp.stack([edge_index_in[1].reshape(NS, NCH, C),
                       edge_index_out[1].reshape(NS, NCH, C)])
    w_b = jnp.stack([edge_weight_in.reshape(NS, PT),
                     edge_weight_out.reshape(NS, PT)])
    zeros = jnp.zeros((RPT, DH), jnp.float32)

    bmi0r, bmo0r = bmi0.reshape(1, D), bmo0.reshape(1, D)
    bsi0r, bso0r = bsi0.reshape(1, D), bso0.reshape(1, D)
    bmi1r, bmo1r = bmi1.reshape(1, D), bmo1.reshape(1, D)
    bsi1r, bso1r = bsi1.reshape(1, D), bso1.reshape(1, D)
    wdec_pad = jnp.zeros((D, D), jnp.float32).at[:CLS].set(W_dec)
    bdec_pad = jnp.zeros((1, D), jnp.float32).at[0, :CLS].set(b_dec)

    tbl0, sh0 = _mm3(x, Wmi0, Wmo0, Ws0)
    pp0 = _propagate2(src_b, dst_b, w_b, tbl0, zeros)

    tbl1, sh1 = _layer_mid(pp0, sh0, bmi0r, bmo0r, bsi0r,
                           bso0r, Cin0, Cout0, Wmi1, Wmo1, Ws1)
    pp1 = _propagate2(src_b, dst_b, w_b, tbl1, zeros)

    logp_pad, norm_emb = _layer_final(pp1, sh1, bmi1r, bmo1r, bsi1r,
                                      bso1r, Cin1, Cout1, wdec_pad, bdec_pad)
    return (logp_pad[:, :CLS], norm_emb)


# speedup vs baseline: 2.0400x; 1.0881x over previous
import functools

import jax
import jax.numpy as jnp
from jax import lax
from jax.experimental import pallas as pl
from jax.experimental.pallas import tpu as pltpu
from jax.experimental.pallas import tpu_sc as plsc

N = 10000
E = 320000
D = 128
CLS = 10
EPS = 1e-12

NC = 2
NS = 16
DH = D // NC
C = 80
PT = E // NS
NCH = PT // C
NBUF = 5
NPAD = 10240
RPT = NPAD // NS


def _sc_prop_body(src_hbm, dst_hbm, w_hbm, table_hbm, zeros_hbm, out_hbm,
                  src_v, dst_v, w_v, rows_v, acc, sem_g, sem_s):
    c = lax.axis_index("c")
    s = lax.axis_index("s")

    pltpu.sync_copy(zeros_hbm, acc.at[pl.ds(s * RPT, RPT)])

    pltpu.sync_copy(src_hbm.at[s], src_v)
    pltpu.sync_copy(dst_hbm.at[s], dst_v)
    pltpu.sync_copy(w_hbm.at[s], w_v)

    plsc.subcore_barrier()

    def gather_start(j, b):
        pltpu.async_copy(table_hbm.at[c].at[src_v.at[j]], rows_v.at[b],
                         sem_g.at[b])

    def gather_wait(j, b):
        pltpu.make_async_copy(table_hbm.at[c].at[src_v.at[j]], rows_v.at[b],
                              sem_g.at[b]).wait()

    def scatter_start(j, b):
        pltpu.async_copy(rows_v.at[b], acc.at[dst_v.at[j]], sem_s.at[b],
                         add=True)

    def scatter_wait(j, b):
        pltpu.make_async_copy(rows_v.at[b], acc.at[dst_v.at[j]],
                              sem_s.at[b]).wait()

    for b in range(NBUF):
        gather_start(b, b)

    def round_body(r, carry):
        for b in range(NBUF):
            j = r * NBUF + b
            gather_wait(j, b)

            for g in range(C // 16):
                w16 = w_v[pl.ds(j * C + g * 16, 16)]
                for l in range(16):
                    wv = w16.at[jnp.full((16,), l, jnp.int32)].get(
                        mode="promise_in_bounds")
                    for k in range(DH // 16):
                        seg = rows_v[b, g * 16 + l, pl.ds(k * 16, 16)]
                        rows_v[b, g * 16 + l, pl.ds(k * 16, 16)] = seg * wv

            bp = (b - 1) % NBUF
            cond = (j - 1 + NBUF < NCH) if b != 0 else \
                jnp.logical_and(r > 0, j - 1 + NBUF < NCH)

            @pl.when(cond)
            def _(bp=bp, j=j):
                scatter_wait(j - 1, bp)
                gather_start(j - 1 + NBUF, bp)

            scatter_start(j, b)

        return carry

    lax.fori_loop(0, NCH // NBUF, round_body, 0)

    for b in range(NBUF):
        scatter_wait(NCH - NBUF + b, b)

    plsc.subcore_barrier()

    pltpu.sync_copy(acc.at[pl.ds(s * RPT, RPT)],
                    out_hbm.at[c].at[pl.ds(s * RPT, RPT)])


def _propagate(src3, dst3, w3, table, zeros):
    mesh = plsc.VectorSubcoreMesh(core_axis_name="c", subcore_axis_name="s",
                                  num_cores=NC, num_subcores=NS)
    return pl.kernel(
        _sc_prop_body,
        out_type=jax.ShapeDtypeStruct((NC, NPAD, DH), jnp.float32),
        mesh=mesh,
        scratch_types=[
            pltpu.VMEM((NCH, C), jnp.int32),
            pltpu.VMEM((NCH, C), jnp.int32),
            pltpu.VMEM((PT,), jnp.float32),
            pltpu.VMEM((NBUF, C, DH), jnp.float32),
            pltpu.VMEM_SHARED((NPAD, DH), jnp.float32),
            pltpu.SemaphoreType.DMA((NBUF,)),
            pltpu.SemaphoreType.DMA((NBUF,)),
        ],
        compiler_params=pltpu.CompilerParams(use_tc_tiling_on_sc=False),
    )(src3, dst3, w3, table, zeros)


_BR = 1000


def _store_halves(ref, val):
    ref[0] = val[:, :DH]
    ref[1] = val[:, DH:]


def _mm3_body(h_ref, wa_ref, wb_ref, wc_ref, a_ref, b_ref, c_ref):
    h = h_ref[...]
    dn = (((1,), (1,)), ((), ()))
    _store_halves(a_ref, lax.dot_general(h, wa_ref[...], dn,
                                         preferred_element_type=jnp.float32))
    _store_halves(b_ref, lax.dot_general(h, wb_ref[...], dn,
                                         preferred_element_type=jnp.float32))
    c_ref[...] = lax.dot_general(h, wc_ref[...], dn,
                                 preferred_element_type=jnp.float32)


def _mm3(h, wa, wb, wc):
    full = pl.BlockSpec((D, D), lambda i: (0, 0))
    hspec = pl.BlockSpec((NC, _BR, DH), lambda i: (0, i, 0))
    return pl.pallas_call(
        _mm3_body,
        grid=(N // _BR,),
        in_specs=[pl.BlockSpec((_BR, D), lambda i: (i, 0)), full, full, full],
        out_specs=[hspec, hspec, pl.BlockSpec((_BR, D), lambda i: (i, 0))],
        out_shape=[jax.ShapeDtypeStruct((NC, N, DH), jnp.float32)] * 2
        + [jax.ShapeDtypeStruct((N, D), jnp.float32)],
    )(h, wa, wb, wc)


def _combine(pin_ref, pout_ref, sh_ref, bmi_ref, bmo_ref, bsi_ref, bso_ref,
             cin_ref, cout_ref):
    pi = jnp.concatenate([pin_ref[0], pin_ref[1]], axis=1)
    po = jnp.concatenate([pout_ref[0], pout_ref[1]], axis=1)
    sh = sh_ref[...]
    ic = pi + bmi_ref[...] + sh + bsi_ref[...]
    oc = po + bmo_ref[...] + sh + bso_ref[...]
    return jnp.tanh(cin_ref[...] * ic + cout_ref[...] * oc)


def _layer_mid_body(pin_ref, pout_ref, sh_ref, bmi_ref, bmo_ref, bsi_ref,
                    bso_ref, cin_ref, cout_ref, wa_ref, wb_ref, wc_ref,
                    a_ref, b_ref, c_ref):
    h = _combine(pin_ref, pout_ref, sh_ref, bmi_ref, bmo_ref, bsi_ref,
                 bso_ref, cin_ref, cout_ref)
    dn = (((1,), (1,)), ((), ()))
    _store_halves(a_ref, lax.dot_general(h, wa_ref[...], dn,
                                         preferred_element_type=jnp.float32))
    _store_halves(b_ref, lax.dot_general(h, wb_ref[...], dn,
                                         preferred_element_type=jnp.float32))
    c_ref[...] = lax.dot_general(h, wc_ref[...], dn,
                                 preferred_element_type=jnp.float32)


def _combine_specs():
    return [
        pl.BlockSpec((NC, _BR, DH), lambda i: (0, i, 0)),
        pl.BlockSpec((NC, _BR, DH), lambda i: (0, i, 0)),
        pl.BlockSpec((_BR, D), lambda i: (i, 0)),
        pl.BlockSpec((1, D), lambda i: (0, 0)),
        pl.BlockSpec((1, D), lambda i: (0, 0)),
        pl.BlockSpec((1, D), lambda i: (0, 0)),
        pl.BlockSpec((1, D), lambda i: (0, 0)),
        pl.BlockSpec((_BR, 1), lambda i: (i, 0)),
        pl.BlockSpec((_BR, 1), lambda i: (i, 0)),
    ]


def _layer_mid(pin, pout, sh, bmi, bmo, bsi, bso, cin, cout, wa, wb, wc):
    full = pl.BlockSpec((D, D), lambda i: (0, 0))
    hspec = pl.BlockSpec((NC, _BR, DH), lambda i: (0, i, 0))
    return pl.pallas_call(
        _layer_mid_body,
        grid=(N // _BR,),
        in_specs=_combine_specs() + [full, full, full],
        out_specs=[hspec, hspec, pl.BlockSpec((_BR, D), lambda i: (i, 0))],
        out_shape=[jax.ShapeDtypeStruct((NC, N, DH), jnp.float32)] * 2
        + [jax.ShapeDtypeStruct((N, D), jnp.float32)],
    )(pin, pout, sh, bmi, bmo, bsi, bso, cin, cout, wa, wb, wc)


def _layer_final_body(pin_ref, pout_ref, sh_ref, bmi_ref, bmo_ref, bsi_ref,
                      bso_ref, cin_ref, cout_ref, wdec_ref, bdec_ref,
                      logp_ref, norm_ref):
    h = _combine(pin_ref, pout_ref, sh_ref, bmi_ref, bmo_ref, bsi_ref,
                 bso_ref, cin_ref, cout_ref)
    s2 = jnp.sum(h * h, axis=1, keepdims=True)
    norm_ref[...] = h / (jnp.sqrt(s2) + EPS)
    dn = (((1,), (1,)), ((), ()))
    logits = lax.dot_general(h, wdec_ref[...], dn,
                             preferred_element_type=jnp.float32)
    logits = logits + bdec_ref[...]
    mask = lax.broadcasted_iota(jnp.int32, (_BR, D), 1) < CLS
    masked = jnp.where(mask, logits, -jnp.inf)
    m = jnp.max(masked, axis=1, keepdims=True)
    ex = jnp.where(mask, jnp.exp(logits - m), 0.0)
    ssum = jnp.sum(ex, axis=1, keepdims=True)
    logp_ref[...] = logits - m - jnp.log(ssum)


def _layer_final(pin, pout, sh, bmi, bmo, bsi, bso, cin, cout, wdec, bdec):
    full = pl.BlockSpec((D, D), lambda i: (0, 0))
    bspec = pl.BlockSpec((1, D), lambda i: (0, 0))
    return pl.pallas_call(
        _layer_final_body,
        grid=(N // _BR,),
        in_specs=_combine_specs() + [full, bspec],
        out_specs=[pl.BlockSpec((_BR, D), lambda i: (i, 0))] * 2,
        out_shape=[jax.ShapeDtypeStruct((N, D), jnp.float32)] * 2,
    )(pin, pout, sh, bmi, bmo, bsi, bso, cin, cout, wdec, bdec)


def kernel(x, edge_index_in, edge_weight_in, edge_index_out, edge_weight_out,
           Wmi0, Wmo0, Ws0, bmi0, bmo0, bsi0, bso0, Cin0, Cout0,
           Wmi1, Wmo1, Ws1, bmi1, bmo1, bsi1, bso1, Cin1, Cout1,
           W_dec, b_dec):
    src_in = edge_index_in[0].reshape(NS, NCH, C)
    dst_in = edge_index_in[1].reshape(NS, NCH, C)
    w_in = edge_weight_in.reshape(NS, PT)
    src_out = edge_index_out[0].reshape(NS, NCH, C)
    dst_out = edge_index_out[1].reshape(NS, NCH, C)
    w_out = edge_weight_out.reshape(NS, PT)
    zeros = jnp.zeros((RPT, DH), jnp.float32)

    bmi0r, bmo0r = bmi0.reshape(1, D), bmo0.reshape(1, D)
    bsi0r, bso0r = bsi0.reshape(1, D), bso0.reshape(1, D)
    bmi1r, bmo1r = bmi1.reshape(1, D), bmo1.reshape(1, D)
    bsi1r, bso1r = bsi1.reshape(1, D), bso1.reshape(1, D)
    wdec_pad = jnp.zeros((D, D), jnp.float32).at[:CLS].set(W_dec)
    bdec_pad = jnp.zeros((1, D), jnp.float32).at[0, :CLS].set(b_dec)

    hmi0, hmo0, sh0 = _mm3(x, Wmi0, Wmo0, Ws0)
    pin0 = _propagate(src_in, dst_in, w_in, hmi0, zeros)
    pout0 = _propagate(src_out, dst_out, w_out, hmo0, zeros)

    hmi1, hmo1, sh1 = _layer_mid(pin0, pout0, sh0, bmi0r, bmo0r, bsi0r,
                                 bso0r, Cin0, Cout0, Wmi1, Wmo1, Ws1)
    pin1 = _propagate(src_in, dst_in, w_in, hmi1, zeros)
    pout1 = _propagate(src_out, dst_out, w_out, hmo1, zeros)

    logp_pad, norm_emb = _layer_final(pin1, pout1, sh1, bmi1r, bmo1r, bsi1r,
                                      bso1r, Cin1, Cout1, wdec_pad, bdec_pad)
    return (logp_pad[:, :CLS], norm_emb)
